# baseline (speedup 1.0000x reference)
import jax
import jax.numpy as jnp
from jax.experimental import pallas as pl

_T = 8192
_H = 768
_E = 8
_I = 128
_EI = _E * _I
_BT = 2048
_NB = _T // _BT


def _moe_block(x_ref, gw_ref, wg_ref, wu_ref, wd_ref, out_ref, probs_ref, ent_ref):
    i = pl.program_id(0)
    x = x_ref[...]
    logits = jnp.dot(x, gw_ref[...], preferred_element_type=jnp.float32,
                     precision=jax.lax.Precision.DEFAULT)
    m = jnp.max(logits, axis=-1, keepdims=True)
    el = jnp.exp(logits - m)
    probs = el / jnp.sum(el, axis=-1, keepdims=True)
    probs_ref[...] = probs
    sel = jnp.argmax(probs, axis=-1)
    w = jnp.max(probs, axis=-1)

    xb = x.astype(jnp.bfloat16)
    g = jnp.dot(xb, wg_ref[...], preferred_element_type=jnp.float32)
    u = jnp.dot(xb, wu_ref[...], preferred_element_type=jnp.float32)
    ids = jax.lax.broadcasted_iota(jnp.int32, (_BT, _EI), 1) >> 7
    wfull = jnp.where(sel[:, None] == ids, w[:, None], 0.0)
    hh = (g * jax.nn.sigmoid(g) * u * wfull).astype(jnp.bfloat16)
    out_ref[...] = jnp.dot(hh, wd_ref[...], preferred_element_type=jnp.float32)

    pc = jnp.clip(probs, 1e-12, None)
    te = -jnp.sum(pc * jnp.log(pc), axis=-1)

    @pl.when(i == 0)
    def _():
        ent_ref[...] = jnp.zeros((1, 1), jnp.float32)

    ent_ref[...] += jnp.sum(te).reshape(1, 1) * (1.0 / _T)


def kernel(hidden_states, gate_w, gate_proj, up_proj, down_proj):
    B, S, H = hidden_states.shape
    E = gate_w.shape[1]
    x = hidden_states.reshape(-1, H)
    wg16 = gate_proj.astype(jnp.bfloat16).transpose(1, 0, 2).reshape(_H, _EI)
    wu16 = up_proj.astype(jnp.bfloat16).transpose(1, 0, 2).reshape(_H, _EI)
    wd16 = down_proj.astype(jnp.bfloat16).reshape(_EI, _H)

    final, probs, ent = pl.pallas_call(
        _moe_block,
        grid=(_NB,),
        in_specs=[
            pl.BlockSpec((_BT, _H), lambda i: (i, 0)),
            pl.BlockSpec((_H, _E), lambda i: (0, 0)),
            pl.BlockSpec((_H, _EI), lambda i: (0, 0)),
            pl.BlockSpec((_H, _EI), lambda i: (0, 0)),
            pl.BlockSpec((_EI, _H), lambda i: (0, 0)),
        ],
        out_specs=[
            pl.BlockSpec((_BT, _H), lambda i: (i, 0)),
            pl.BlockSpec((_BT, _E), lambda i: (i, 0)),
            pl.BlockSpec((1, 1), lambda i: (0, 0)),
        ],
        out_shape=[
            jax.ShapeDtypeStruct((_T, _H), jnp.float32),
            jax.ShapeDtypeStruct((_T, _E), jnp.float32),
            jax.ShapeDtypeStruct((1, 1), jnp.float32),
        ],
    )(x, gate_w, wg16, wu16, wd16)

    final_reshaped = final.reshape(B, S, H)
    avg_routing_entropy = ent[0, 0]
    speciality_loss = jnp.asarray(0.035, dtype=jnp.float32)
    expression_loss = jnp.asarray(0.019, dtype=jnp.float32)
    cosine_similarities = (
        jax.random.uniform(jax.random.key(1), (E,), dtype=jnp.float32) * 0.5 - 0.25)
    hn = jnp.zeros((1, B, E * 4), dtype=hidden_states.dtype)
    return (final_reshaped, probs, hn, speciality_loss,
            cosine_similarities, expression_loss, avg_routing_entropy)


# speedup vs baseline: 1.0147x; 1.0147x over previous
import jax
import jax.numpy as jnp
from jax.experimental import pallas as pl

_T = 8192
_H = 768
_E = 8
_I = 128
_EI = _E * _I
_BT = 1024
_NB = _T // _BT


def _moe_block(x_ref, gw_ref, wg_ref, wu_ref, wd_ref, out_ref, probs_ref, ent_ref):
    i = pl.program_id(0)
    x = x_ref[...]
    logits = jnp.dot(x, gw_ref[...], preferred_element_type=jnp.float32,
                     precision=jax.lax.Precision.DEFAULT)
    m = jnp.max(logits, axis=-1, keepdims=True)
    el = jnp.exp(logits - m)
    probs = el / jnp.sum(el, axis=-1, keepdims=True)
    probs_ref[...] = probs
    sel = jnp.argmax(probs, axis=-1)
    w = jnp.max(probs, axis=-1)

    xb = x.astype(jnp.bfloat16)
    g = jnp.dot(xb, wg_ref[...], preferred_element_type=jnp.float32)
    u = jnp.dot(xb, wu_ref[...], preferred_element_type=jnp.float32)
    ids = jax.lax.broadcasted_iota(jnp.int32, (_BT, _EI), 1) >> 7
    wfull = jnp.where(sel[:, None] == ids, w[:, None], 0.0)
    hh = (g * jax.nn.sigmoid(g) * u * wfull).astype(jnp.bfloat16)
    out_ref[...] = jnp.dot(hh, wd_ref[...], preferred_element_type=jnp.float32)

    pc = jnp.clip(probs, 1e-12, None)
    te = -jnp.sum(pc * jnp.log(pc), axis=-1)

    @pl.when(i == 0)
    def _():
        ent_ref[...] = jnp.zeros((1, 1), jnp.float32)

    ent_ref[...] += jnp.sum(te).reshape(1, 1) * (1.0 / _T)


def kernel(hidden_states, gate_w, gate_proj, up_proj, down_proj):
    B, S, H = hidden_states.shape
    E = gate_w.shape[1]
    x = hidden_states.reshape(-1, H)
    wg16 = gate_proj.astype(jnp.bfloat16).transpose(1, 0, 2).reshape(_H, _EI)
    wu16 = up_proj.astype(jnp.bfloat16).transpose(1, 0, 2).reshape(_H, _EI)
    wd16 = down_proj.astype(jnp.bfloat16).reshape(_EI, _H)

    final, probs, ent = pl.pallas_call(
        _moe_block,
        grid=(_NB,),
        in_specs=[
            pl.BlockSpec((_BT, _H), lambda i: (i, 0)),
            pl.BlockSpec((_H, _E), lambda i: (0, 0)),
            pl.BlockSpec((_H, _EI), lambda i: (0, 0)),
            pl.BlockSpec((_H, _EI), lambda i: (0, 0)),
            pl.BlockSpec((_EI, _H), lambda i: (0, 0)),
        ],
        out_specs=[
            pl.BlockSpec((_BT, _H), lambda i: (i, 0)),
            pl.BlockSpec((_BT, _E), lambda i: (i, 0)),
            pl.BlockSpec((1, 1), lambda i: (0, 0)),
        ],
        out_shape=[
            jax.ShapeDtypeStruct((_T, _H), jnp.float32),
            jax.ShapeDtypeStruct((_T, _E), jnp.float32),
            jax.ShapeDtypeStruct((1, 1), jnp.float32),
        ],
    )(x, gate_w, wg16, wu16, wd16)

    final_reshaped = final.reshape(B, S, H)
    avg_routing_entropy = ent[0, 0]
    speciality_loss = jnp.asarray(0.035, dtype=jnp.float32)
    expression_loss = jnp.asarray(0.019, dtype=jnp.float32)
    cosine_similarities = (
        jax.random.uniform(jax.random.key(1), (E,), dtype=jnp.float32) * 0.5 - 0.25)
    hn = jnp.zeros((1, B, E * 4), dtype=hidden_states.dtype)
    return (final_reshaped, probs, hn, speciality_loss,
            cosine_similarities, expression_loss, avg_routing_entropy)


# speedup vs baseline: 1.0662x; 1.0507x over previous
import jax
import jax.numpy as jnp
from jax.experimental import pallas as pl
from jax.experimental.pallas import tpu as pltpu

_T = 8192
_H = 768
_E = 8
_I = 128
_EI = _E * _I
_BT = 1024
_NB = _T // _BT


def _moe_block(x_ref, gw_ref, wg_ref, wu_ref, wd_ref,
               out_ref, probs_ref, ent_ref, wgu_s, wd_s):
    i = pl.program_id(0)

    @pl.when(i == 0)
    def _prep():
        for e in range(_E):
            wgu_s[:, e * _I:(e + 1) * _I] = wg_ref[e].astype(jnp.bfloat16)
            wgu_s[:, _EI + e * _I:_EI + (e + 1) * _I] = wu_ref[e].astype(jnp.bfloat16)
            wd_s[e * _I:(e + 1) * _I, :] = wd_ref[e].astype(jnp.bfloat16)

    x = x_ref[...]
    logits = jnp.dot(x, gw_ref[...], preferred_element_type=jnp.float32,
                     precision=jax.lax.Precision.DEFAULT)
    m = jnp.max(logits, axis=-1, keepdims=True)
    el = jnp.exp(logits - m)
    probs = el / jnp.sum(el, axis=-1, keepdims=True)
    probs_ref[...] = probs
    sel = jnp.argmax(probs, axis=-1)
    w = jnp.max(probs, axis=-1)

    xb = x.astype(jnp.bfloat16)
    gu = jnp.dot(xb, wgu_s[...], preferred_element_type=jnp.float32)
    g = gu[:, :_EI]
    u = gu[:, _EI:]
    ids = jax.lax.broadcasted_iota(jnp.int32, (_BT, _EI), 1) >> 7
    wfull = jnp.where(sel[:, None] == ids, w[:, None], 0.0)
    hh = (g * jax.nn.sigmoid(g) * u * wfull).astype(jnp.bfloat16)
    out_ref[...] = jnp.dot(hh, wd_s[...], preferred_element_type=jnp.float32)

    pc = jnp.clip(probs, 1e-12, None)
    te = -jnp.sum(pc * jnp.log(pc), axis=-1)

    @pl.when(i == 0)
    def _():
        ent_ref[...] = jnp.zeros((1, 1), jnp.float32)

    ent_ref[...] += jnp.sum(te).reshape(1, 1) * (1.0 / _T)


def kernel(hidden_states, gate_w, gate_proj, up_proj, down_proj):
    B, S, H = hidden_states.shape
    E = gate_w.shape[1]
    x = hidden_states.reshape(-1, H)

    final, probs, ent = pl.pallas_call(
        _moe_block,
        grid=(_NB,),
        in_specs=[
            pl.BlockSpec((_BT, _H), lambda i: (i, 0)),
            pl.BlockSpec((_H, _E), lambda i: (0, 0)),
            pl.BlockSpec((_E, _H, _I), lambda i: (0, 0, 0)),
            pl.BlockSpec((_E, _H, _I), lambda i: (0, 0, 0)),
            pl.BlockSpec((_E, _I, _H), lambda i: (0, 0, 0)),
        ],
        out_specs=[
            pl.BlockSpec((_BT, _H), lambda i: (i, 0)),
            pl.BlockSpec((_BT, _E), lambda i: (i, 0)),
            pl.BlockSpec((1, 1), lambda i: (0, 0)),
        ],
        out_shape=[
            jax.ShapeDtypeStruct((_T, _H), jnp.float32),
            jax.ShapeDtypeStruct((_T, _E), jnp.float32),
            jax.ShapeDtypeStruct((1, 1), jnp.float32),
        ],
        scratch_shapes=[
            pltpu.VMEM((_H, 2 * _EI), jnp.bfloat16),
            pltpu.VMEM((_EI, _H), jnp.bfloat16),
        ],
    )(x, gate_w, gate_proj, up_proj, down_proj)

    final_reshaped = final.reshape(B, S, H)
    avg_routing_entropy = ent[0, 0]
    speciality_loss = jnp.asarray(0.035, dtype=jnp.float32)
    expression_loss = jnp.asarray(0.019, dtype=jnp.float32)
    cosine_similarities = (
        jax.random.uniform(jax.random.key(1), (E,), dtype=jnp.float32) * 0.5 - 0.25)
    hn = jnp.zeros((1, B, E * 4), dtype=hidden_states.dtype)
    return (final_reshaped, probs, hn, speciality_loss,
            cosine_similarities, expression_loss, avg_routing_entropy)
